# baseline (speedup 1.0000x reference)
import functools

import jax
import jax.numpy as jnp
from jax import lax
from jax.experimental import pallas as pl
from jax.experimental.pallas import tpu as pltpu
from jax.experimental.pallas import tpu_sc as plsc

N = 10000
E = 320000
D = 128
NC = 2
NS = 16
NW = NC * NS
CHUNK = 128
NCHUNKS = E // CHUNK
NPW = (NCHUNKS + NW - 1) // NW
NITER = (NPW + 11) // 12 * 12
EP = NW * NPW * CHUNK
ACC = 10104
W0 = 632
PADY = 8
_PIECES0 = [(0, 128), (128, 128), (256, 128), (384, 128), (512, 120)]
_PIECES15 = [(0, 128), (128, 128), (256, 128), (384, 128), (512, 8)]

_mesh = plsc.VectorSubcoreMesh(core_axis_name="c", subcore_axis_name="s")


@functools.partial(
    pl.kernel,
    out_type=jax.ShapeDtypeStruct((NC, N, D), jnp.float32),
    mesh=_mesh,
    scratch_types=[
        pltpu.VMEM((2, CHUNK), jnp.int32),
        pltpu.VMEM((2, CHUNK), jnp.int32),
        pltpu.VMEM((2, CHUNK), jnp.int32),
        pltpu.VMEM((2, CHUNK), jnp.int32),
        pltpu.VMEM((CHUNK, D), jnp.float32),
        pltpu.VMEM((CHUNK, D), jnp.float32),
        pltpu.VMEM((CHUNK, D), jnp.float32),
        pltpu.VMEM_SHARED((ACC, D), jnp.float32),
        pltpu.SemaphoreType.DMA,
        pltpu.SemaphoreType.DMA,
        pltpu.SemaphoreType.DMA,
        pltpu.SemaphoreType.DMA,
        pltpu.SemaphoreType.DMA,
        pltpu.SemaphoreType.DMA,
        pltpu.SemaphoreType.DMA,
        pltpu.SemaphoreType.DMA,
        pltpu.SemaphoreType.DMA,
        pltpu.SemaphoreType.DMA,
    ],
)
def _agg_kernel(y_hbm, ei_hbm, out_hbm,
                i0, i1, i2, i3, ra, rb, rc,
                agg, is0, is1, is2, is3, ga, gb, gc, pa, pb, pc):
    cid = lax.axis_index("c")
    sid = lax.axis_index("s")
    wid = cid * NS + sid

    IDX = [i0, i1, i2, i3]
    ISEM = [is0, is1, is2, is3]
    ROWS = [ra, rb, rc]
    GSEM = [ga, gb, gc]
    SSEM = [pa, pb, pc]

    @pl.loop(0, CHUNK)
    def _(i):
        for j in range(D // 16):
            ra[i, pl.ds(j * 16, 16)] = jnp.zeros((16,), jnp.float32)

    @pl.when(sid < NS - 1)
    def _():
        for off, sz in _PIECES0:
            pltpu.sync_copy(
                ra.at[pl.ds(0, sz), :],
                agg.at[pl.ds(sid * W0 + off, sz), :])

    @pl.when(sid == NS - 1)
    def _():
        for off, sz in _PIECES15:
            pltpu.sync_copy(
                ra.at[pl.ds(0, sz), :],
                agg.at[pl.ds(sid * W0 + off, sz), :])

    plsc.subcore_barrier()

    def guarded(c, fn):
        @pl.when(jnp.logical_and(c >= 0, c < NPW))
        def _():
            fn()

    def load_idx(j, s):
        guarded(j, lambda: pltpu.async_copy(
            ei_hbm.at[:, pl.ds((wid * NPW + j) * CHUNK, CHUNK)],
            IDX[s], ISEM[s]))

    def gth(j, s, m):
        def fn():
            pltpu.make_async_copy(
                ei_hbm.at[:, pl.ds((wid * NPW + j) * CHUNK, CHUNK)],
                IDX[s], ISEM[s]).wait()
            pltpu.async_copy(y_hbm.at[IDX[s].at[1]], ROWS[m], GSEM[m])
        guarded(j, fn)

    def sct(j, s, m):
        def fn():
            pltpu.make_async_copy(
                y_hbm.at[IDX[s].at[1]], ROWS[m], GSEM[m]).wait()
            pltpu.async_copy(ROWS[m], agg.at[IDX[s].at[0]], SSEM[m], add=True)
        guarded(j, fn)

    def wait_sct(j, s, m):
        guarded(j, lambda: pltpu.make_async_copy(
            ROWS[m], agg.at[IDX[s].at[0]], SSEM[m]).wait())

    load_idx(0, 0)
    load_idx(1, 1)
    load_idx(2, 2)
    gth(0, 0, 0)
    gth(1, 1, 1)

    @pl.loop(0, NITER, step=12)
    def _(i):
        for k in range(12):
            j = i + k
            wait_sct(j - 1, (k + 3) % 4, (k + 2) % 3)
            load_idx(j + 3, (k + 3) % 4)
            gth(j + 2, (k + 2) % 4, (k + 2) % 3)
            sct(j, k % 4, k % 3)

    plsc.subcore_barrier()

    @pl.when(sid < NS - 1)
    def _():
        for off, sz in _PIECES0:
            pltpu.sync_copy(
                agg.at[pl.ds(sid * W0 + off, sz), :],
                out_hbm.at[cid, pl.ds(sid * W0 + off, sz), :])

    @pl.when(sid == NS - 1)
    def _():
        for off, sz in _PIECES15:
            pltpu.sync_copy(
                agg.at[pl.ds(sid * W0 + off, sz), :],
                out_hbm.at[cid, pl.ds(sid * W0 + off, sz), :])


_BN = 1000
_BE = 12800
_NEB = E // _BE


def _hist_body(rows_ref, o_ref):
    r = rows_ref[0]
    hi = jnp.right_shift(r, 7)
    lo = jnp.bitwise_and(r, 127)
    ids = lax.broadcasted_iota(jnp.int32, (128, _BE), 0)
    oh_hi = (ids == hi).astype(jnp.bfloat16)
    oh_lo = (ids == lo).astype(jnp.bfloat16)
    contrib = lax.dot_general(
        oh_hi, oh_lo, (((1,), (1,)), ((), ())),
        preferred_element_type=jnp.float32)

    @pl.when(pl.program_id(0) == 0)
    def _():
        o_ref[...] = jnp.zeros_like(o_ref)

    o_ref[...] += contrib


def _dinv_from(deg_ref):
    d = deg_ref[...]
    return jnp.where(d > 0, lax.rsqrt(jnp.where(d > 0, d, 1.0)), 0.0)


def _mmscale_body(x_ref, w_ref, deg_ref, y_ref):
    xw = lax.dot_general(
        x_ref[...], w_ref[...], (((1,), (1,)), ((), ())),
        precision=lax.Precision.HIGHEST, preferred_element_type=jnp.float32)
    y_ref[...] = xw * _dinv_from(deg_ref)


def _final_body(parts_ref, deg_ref, b_ref, o_ref):
    s = parts_ref[0] + parts_ref[1]
    o_ref[...] = s * _dinv_from(deg_ref) + b_ref[...]


def kernel(x, edge_index, W, b):
    f32 = jnp.float32

    rows3 = edge_index[0].reshape(_NEB, 1, _BE)
    deg2d = pl.pallas_call(
        _hist_body,
        grid=(_NEB,),
        in_specs=[pl.BlockSpec((1, 1, _BE), lambda i: (i, 0, 0))],
        out_specs=pl.BlockSpec((128, 128), lambda i: (0, 0)),
        out_shape=jax.ShapeDtypeStruct((128, 128), f32),
    )(rows3)
    deg_col = deg2d.reshape(128 * 128, 1)[:N]

    y = pl.pallas_call(
        _mmscale_body,
        grid=(N // _BN,),
        in_specs=[
            pl.BlockSpec((_BN, D), lambda i: (i, 0)),
            pl.BlockSpec((D, D), lambda i: (0, 0)),
            pl.BlockSpec((_BN, 1), lambda i: (i, 0)),
        ],
        out_specs=pl.BlockSpec((_BN, D), lambda i: (i, 0)),
        out_shape=jax.ShapeDtypeStruct((N, D), f32),
    )(x, W, deg_col)

    pad = jnp.concatenate(
        [jnp.zeros((1, EP - E), jnp.int32),
         jnp.full((1, EP - E), N, jnp.int32)], axis=0)
    ei_p = jnp.concatenate([edge_index, pad], axis=1)
    y_p = jnp.pad(y, ((0, PADY), (0, 0)))

    parts = _agg_kernel(y_p, ei_p)

    out = pl.pallas_call(
        _final_body,
        grid=(N // _BN,),
        in_specs=[
            pl.BlockSpec((NC, _BN, D), lambda i: (0, i, 0)),
            pl.BlockSpec((_BN, 1), lambda i: (i, 0)),
            pl.BlockSpec((1, D), lambda i: (0, 0)),
        ],
        out_specs=pl.BlockSpec((_BN, D), lambda i: (i, 0)),
        out_shape=jax.ShapeDtypeStruct((N, D), f32),
    )(parts, deg_col, b.reshape(1, D))

    return out


# speedup vs baseline: 2.1055x; 2.1055x over previous
import functools

import jax
import jax.numpy as jnp
from jax import lax
from jax.experimental import pallas as pl
from jax.experimental.pallas import tpu as pltpu
from jax.experimental.pallas import tpu_sc as plsc

N = 10000
E = 320000
D = 128
NC = 2
NS = 16
NW = NC * NS
CHUNK = 128
NCHUNKS = E // CHUNK
NITER = (NCHUNKS + NW - 1) // NW + 1
NPAD = 10240
RPT = NPAD // NS

_mesh = plsc.VectorSubcoreMesh(core_axis_name="c", subcore_axis_name="s")


@functools.partial(
    pl.kernel,
    out_type=jax.ShapeDtypeStruct((NC, NPAD, D), jnp.float32),
    mesh=_mesh,
    scratch_types=[
        pltpu.VMEM((2, CHUNK), jnp.int32),
        pltpu.VMEM((2, CHUNK), jnp.int32),
        pltpu.VMEM((2, CHUNK), jnp.int32),
        pltpu.VMEM((2, CHUNK), jnp.int32),
        pltpu.VMEM((CHUNK, D), jnp.float32),
        pltpu.VMEM((CHUNK, D), jnp.float32),
        pltpu.VMEM_SHARED((NPAD, D), jnp.float32),
        pltpu.SemaphoreType.DMA,
        pltpu.SemaphoreType.DMA,
        pltpu.SemaphoreType.DMA,
        pltpu.SemaphoreType.DMA,
        pltpu.SemaphoreType.DMA,
        pltpu.SemaphoreType.DMA,
    ],
)
def _agg_kernel(y_hbm, ei_hbm, out_hbm,
                idxa, idxb, idxc, idxd, rows0, rows1,
                agg, sem0, sem1, isema, isemb, isemc, isemd):
    cid = lax.axis_index("c")
    sid = lax.axis_index("s")
    wid = cid * NS + sid

    @pl.loop(0, CHUNK)
    def _(i):
        for j in range(D // 16):
            rows0[i, pl.ds(j * 16, 16)] = jnp.zeros((16,), jnp.float32)

    for k in range(RPT // CHUNK):
        pltpu.sync_copy(rows0, agg.at[pl.ds(sid * RPT + k * CHUNK, CHUNK), :])
    plsc.subcore_barrier()

    def load_idx(i, idx, isem):
        c = i * NW + wid

        @pl.when(c < NCHUNKS)
        def _():
            pltpu.async_copy(ei_hbm.at[:, pl.ds(c * CHUNK, CHUNK)], idx, isem)

    def gather(i, idx, rows, isem, sem):
        c = i * NW + wid

        @pl.when(c < NCHUNKS)
        def _():
            pltpu.make_async_copy(
                ei_hbm.at[:, pl.ds(c * CHUNK, CHUNK)], idx, isem).wait()
            pltpu.async_copy(y_hbm.at[idx.at[1]], rows, sem)

    def scatter(i, idx, rows, sem):
        c = i * NW + wid

        @pl.when(c < NCHUNKS)
        def _():
            pltpu.make_async_copy(y_hbm.at[idx.at[1]], rows, sem).wait()
            pltpu.sync_copy(rows, agg.at[idx.at[0]], add=True)

    load_idx(0, idxa, isema)
    load_idx(1, idxb, isemb)
    load_idx(2, idxc, isemc)
    load_idx(3, idxd, isemd)
    gather(0, idxa, rows0, isema, sem0)

    @pl.loop(0, NITER, step=4)
    def _(i):
        gather(i + 1, idxb, rows1, isemb, sem1)
        scatter(i, idxa, rows0, sem0)
        load_idx(i + 4, idxa, isema)
        gather(i + 2, idxc, rows0, isemc, sem0)
        scatter(i + 1, idxb, rows1, sem1)
        load_idx(i + 5, idxb, isemb)
        gather(i + 3, idxd, rows1, isemd, sem1)
        scatter(i + 2, idxc, rows0, sem0)
        load_idx(i + 6, idxc, isemc)
        gather(i + 4, idxa, rows0, isema, sem0)
        scatter(i + 3, idxd, rows1, sem1)
        load_idx(i + 7, idxd, isemd)

    plsc.subcore_barrier()
    pltpu.sync_copy(
        agg.at[pl.ds(sid * RPT, RPT), :],
        out_hbm.at[cid, pl.ds(sid * RPT, RPT), :],
    )


_BN = 1000
_BE = 12800
_NEB = E // _BE


def _hist_body(rows_ref, o_ref):
    r = rows_ref[0]
    hi = jnp.right_shift(r, 7)
    lo = jnp.bitwise_and(r, 127)
    ids = lax.broadcasted_iota(jnp.int32, (128, _BE), 0)
    oh_hi = (ids == hi).astype(jnp.bfloat16)
    oh_lo = (ids == lo).astype(jnp.bfloat16)
    contrib = lax.dot_general(
        oh_hi, oh_lo, (((1,), (1,)), ((), ())),
        preferred_element_type=jnp.float32)

    @pl.when(pl.program_id(0) == 0)
    def _():
        o_ref[...] = jnp.zeros_like(o_ref)

    o_ref[...] += contrib


def _dinv_from(deg_ref):
    d = deg_ref[...]
    return jnp.where(d > 0, lax.rsqrt(jnp.where(d > 0, d, 1.0)), 0.0)


def _mmscale_body(x_ref, w_ref, deg_ref, y_ref):
    xw = lax.dot_general(
        x_ref[...], w_ref[...], (((1,), (1,)), ((), ())),
        precision=lax.Precision.HIGHEST, preferred_element_type=jnp.float32)
    y_ref[...] = xw * _dinv_from(deg_ref)


def _final_body(parts_ref, deg_ref, b_ref, o_ref):
    s = parts_ref[0] + parts_ref[1]
    o_ref[...] = s * _dinv_from(deg_ref) + b_ref[...]


def kernel(x, edge_index, W, b):
    f32 = jnp.float32

    rows3 = edge_index[0].reshape(_NEB, 1, _BE)
    deg2d = pl.pallas_call(
        _hist_body,
        grid=(_NEB,),
        in_specs=[pl.BlockSpec((1, 1, _BE), lambda i: (i, 0, 0))],
        out_specs=pl.BlockSpec((128, 128), lambda i: (0, 0)),
        out_shape=jax.ShapeDtypeStruct((128, 128), f32),
    )(rows3)
    deg_col = deg2d.reshape(128 * 128, 1)[:N]

    y = pl.pallas_call(
        _mmscale_body,
        grid=(N // _BN,),
        in_specs=[
            pl.BlockSpec((_BN, D), lambda i: (i, 0)),
            pl.BlockSpec((D, D), lambda i: (0, 0)),
            pl.BlockSpec((_BN, 1), lambda i: (i, 0)),
        ],
        out_specs=pl.BlockSpec((_BN, D), lambda i: (i, 0)),
        out_shape=jax.ShapeDtypeStruct((N, D), f32),
    )(x, W, deg_col)

    parts = _agg_kernel(y, edge_index)

    out = pl.pallas_call(
        _final_body,
        grid=(N // _BN,),
        in_specs=[
            pl.BlockSpec((NC, _BN, D), lambda i: (0, i, 0)),
            pl.BlockSpec((_BN, 1), lambda i: (i, 0)),
            pl.BlockSpec((1, D), lambda i: (0, 0)),
        ],
        out_specs=pl.BlockSpec((_BN, D), lambda i: (i, 0)),
        out_shape=jax.ShapeDtypeStruct((N, D), f32),
    )(parts, deg_col, b.reshape(1, D))

    return out


# speedup vs baseline: 2.1422x; 1.0174x over previous
import functools

import jax
import jax.numpy as jnp
from jax import lax
from jax.experimental import pallas as pl
from jax.experimental.pallas import tpu as pltpu
from jax.experimental.pallas import tpu_sc as plsc

N = 10000
E = 320000
D = 128
NC = 2
NS = 16
NW = NC * NS
CHUNK = 128
NCHUNKS = E // CHUNK
NITER = (NCHUNKS + NW - 1) // NW + 1
NPAD = 10240
RPT = NPAD // NS

_mesh = plsc.VectorSubcoreMesh(core_axis_name="c", subcore_axis_name="s")


@functools.partial(
    pl.kernel,
    out_type=jax.ShapeDtypeStruct((NC, NPAD, D), jnp.float32),
    mesh=_mesh,
    scratch_types=[
        pltpu.VMEM((2, CHUNK), jnp.int32),
        pltpu.VMEM((2, CHUNK), jnp.int32),
        pltpu.VMEM((2, CHUNK), jnp.int32),
        pltpu.VMEM((2, CHUNK), jnp.int32),
        pltpu.VMEM((CHUNK, D), jnp.float32),
        pltpu.VMEM((CHUNK, D), jnp.float32),
        pltpu.VMEM_SHARED((NPAD, D), jnp.float32),
        pltpu.SemaphoreType.DMA,
        pltpu.SemaphoreType.DMA,
        pltpu.SemaphoreType.DMA,
        pltpu.SemaphoreType.DMA,
        pltpu.SemaphoreType.DMA,
        pltpu.SemaphoreType.DMA,
    ],
)
def _agg_kernel(y_hbm, ei_hbm, out_hbm,
                idxa, idxb, idxc, idxd, rows0, rows1,
                agg, sem0, sem1, isema, isemb, isemc, isemd):
    cid = lax.axis_index("c")
    sid = lax.axis_index("s")
    wid = cid * NS + sid

    @pl.loop(0, CHUNK)
    def _(i):
        for j in range(D // 16):
            rows0[i, pl.ds(j * 16, 16)] = jnp.zeros((16,), jnp.float32)

    for k in range(RPT // CHUNK):
        pltpu.sync_copy(rows0, agg.at[pl.ds(sid * RPT + k * CHUNK, CHUNK), :])
    plsc.subcore_barrier()

    def load_idx(i, idx, isem):
        c = i * NW + wid

        @pl.when(c < NCHUNKS)
        def _():
            pltpu.async_copy(ei_hbm.at[:, pl.ds(c * CHUNK, CHUNK)], idx, isem)

    def gather(i, idx, rows, isem, sem):
        c = i * NW + wid

        @pl.when(c < NCHUNKS)
        def _():
            pltpu.make_async_copy(
                ei_hbm.at[:, pl.ds(c * CHUNK, CHUNK)], idx, isem).wait()
            pltpu.async_copy(y_hbm.at[idx.at[1]], rows, sem)

    def scatter(i, idx, rows, sem):
        c = i * NW + wid

        @pl.when(c < NCHUNKS)
        def _():
            pltpu.make_async_copy(y_hbm.at[idx.at[1]], rows, sem).wait()
            pltpu.sync_copy(rows, agg.at[idx.at[0]], add=True)

    load_idx(0, idxa, isema)
    load_idx(1, idxb, isemb)
    load_idx(2, idxc, isemc)
    load_idx(3, idxd, isemd)
    gather(0, idxa, rows0, isema, sem0)

    @pl.loop(0, NITER, step=4)
    def _(i):
        gather(i + 1, idxb, rows1, isemb, sem1)
        scatter(i, idxa, rows0, sem0)
        load_idx(i + 4, idxa, isema)
        gather(i + 2, idxc, rows0, isemc, sem0)
        scatter(i + 1, idxb, rows1, sem1)
        load_idx(i + 5, idxb, isemb)
        gather(i + 3, idxd, rows1, isemd, sem1)
        scatter(i + 2, idxc, rows0, sem0)
        load_idx(i + 6, idxc, isemc)
        gather(i + 4, idxa, rows0, isema, sem0)
        scatter(i + 3, idxd, rows1, sem1)
        load_idx(i + 7, idxd, isemd)

    plsc.subcore_barrier()
    pltpu.sync_copy(
        agg.at[pl.ds(sid * RPT, RPT), :],
        out_hbm.at[cid, pl.ds(sid * RPT, RPT), :],
    )


_BN = 1000
_BE = 32000
_NEB = E // _BE


def _hist_body(rows_ref, o_ref):
    r = rows_ref[0]
    hi = jnp.right_shift(r, 7)
    lo = jnp.bitwise_and(r, 127)
    ids = lax.broadcasted_iota(jnp.int32, (128, _BE), 0)
    oh_hi = (ids == hi).astype(jnp.bfloat16)
    oh_lo = (ids == lo).astype(jnp.bfloat16)
    contrib = lax.dot_general(
        oh_hi, oh_lo, (((1,), (1,)), ((), ())),
        preferred_element_type=jnp.float32)

    @pl.when(pl.program_id(0) == 0)
    def _():
        o_ref[...] = jnp.zeros_like(o_ref)

    o_ref[...] += contrib


def _dinv_from(deg_ref):
    d = deg_ref[...]
    return jnp.where(d > 0, lax.rsqrt(jnp.where(d > 0, d, 1.0)), 0.0)


def _mmscale_body(x_ref, w_ref, deg_ref, y_ref):
    xw = lax.dot_general(
        x_ref[...], w_ref[...], (((1,), (1,)), ((), ())),
        precision=lax.Precision.HIGHEST, preferred_element_type=jnp.float32)
    y_ref[...] = xw * _dinv_from(deg_ref)


def _final_body(parts_ref, deg_ref, b_ref, o_ref):
    s = parts_ref[0] + parts_ref[1]
    o_ref[...] = s * _dinv_from(deg_ref) + b_ref[...]


def kernel(x, edge_index, W, b):
    f32 = jnp.float32

    rows3 = edge_index[0].reshape(_NEB, 1, _BE)
    deg2d = pl.pallas_call(
        _hist_body,
        grid=(_NEB,),
        in_specs=[pl.BlockSpec((1, 1, _BE), lambda i: (i, 0, 0))],
        out_specs=pl.BlockSpec((128, 128), lambda i: (0, 0)),
        out_shape=jax.ShapeDtypeStruct((128, 128), f32),
    )(rows3)
    deg_col = deg2d.reshape(128 * 128, 1)[:N]

    y = pl.pallas_call(
        _mmscale_body,
        grid=(N // _BN,),
        in_specs=[
            pl.BlockSpec((_BN, D), lambda i: (i, 0)),
            pl.BlockSpec((D, D), lambda i: (0, 0)),
            pl.BlockSpec((_BN, 1), lambda i: (i, 0)),
        ],
        out_specs=pl.BlockSpec((_BN, D), lambda i: (i, 0)),
        out_shape=jax.ShapeDtypeStruct((N, D), f32),
    )(x, W, deg_col)

    parts = _agg_kernel(y, edge_index)

    out = pl.pallas_call(
        _final_body,
        grid=(N // _BN,),
        in_specs=[
            pl.BlockSpec((NC, _BN, D), lambda i: (0, i, 0)),
            pl.BlockSpec((_BN, 1), lambda i: (i, 0)),
            pl.BlockSpec((1, D), lambda i: (0, 0)),
        ],
        out_specs=pl.BlockSpec((_BN, D), lambda i: (i, 0)),
        out_shape=jax.ShapeDtypeStruct((N, D), f32),
    )(parts, deg_col, b.reshape(1, D))

    return out
